# baseline (speedup 1.0000x reference)
import jax
import jax.numpy as jnp
from jax import lax
from jax.experimental import pallas as pl
from jax.experimental.pallas import tpu as pltpu
from jax.experimental.pallas import tpu_sc as plsc

BATCH = 16384
FACTOR = 32
KROWS = 1000000
ROWS_PER_LINE = 4
NLINES = KROWS // ROWS_PER_LINE
LINE = ROWS_PER_LINE * FACTOR
NC = 2
NS = 16
NW = NC * NS
B_PER_W = BATCH // NW
CHUNK = 128
NCHUNK = B_PER_W // CHUNK
NBUF = 2

TCOLS = 2048
TLINES = TCOLS // ROWS_PER_LINE
TGRID = (KROWS + TCOLS - 1) // TCOLS


def _tbody(x_ref, eye_ref, o_ref):
    eye = eye_ref[...]
    parts = []
    for q in range(ROWS_PER_LINE):
        xq = x_ref[:, q * TLINES:(q + 1) * TLINES]
        parts.append(lax.dot_general(
            xq, eye, (((0,), (0,)), ((), ())),
            preferred_element_type=jnp.float32))
    o_ref[...] = jnp.concatenate(parts, axis=1)


_tc_lines = pl.pallas_call(
    _tbody,
    grid=(TGRID,),
    in_specs=[pl.BlockSpec((FACTOR, TCOLS), lambda i: (0, i)),
              pl.BlockSpec((FACTOR, FACTOR), lambda i: (0, 0))],
    out_specs=pl.BlockSpec((TLINES, LINE), lambda i: (i, 0)),
    out_shape=jax.ShapeDtypeStruct((TGRID * TLINES, LINE), jnp.float32),
)


def _body(table_hbm, fromk_hbm, tok_hbm, wb_hbm, out_hbm,
          idx_a, idx_b, line_a, line_b, rows_a, rows_b, wb_v, out_v,
          sem_i, sem_a, sem_b):
    wid = lax.axis_index("s") * NC + lax.axis_index("c")
    base = wid * B_PER_W

    cp_a = pltpu.async_copy(fromk_hbm.at[pl.ds(base, B_PER_W)], idx_a,
                            sem_i)
    cp_b = pltpu.async_copy(tok_hbm.at[pl.ds(base, B_PER_W)], idx_b,
                            sem_i)
    pltpu.sync_copy(wb_hbm, wb_v)
    cp_a.wait()
    cp_b.wait()

    @plsc.parallel_loop(0, B_PER_W, step=16, unroll=4)
    def _shift(k):
        iva = idx_a[pl.ds(k, 16)]
        ivb = idx_b[pl.ds(k, 16)]
        line_a[pl.ds(k, 16)] = (
            lax.shift_left(lax.shift_right_logical(iva, 11), 9) + (iva & 511))
        line_b[pl.ds(k, 16)] = (
            lax.shift_left(lax.shift_right_logical(ivb, 11), 9) + (ivb & 511))

    def fire(j):
        buf = j % NBUF
        ids = pl.ds(j * CHUNK, CHUNK)
        return (
            pltpu.async_copy(table_hbm.at[line_a.at[ids]], rows_a.at[buf],
                             sem_a),
            pltpu.async_copy(table_hbm.at[line_b.at[ids]], rows_b.at[buf],
                             sem_b),
        )

    pending = fire(0)

    w0 = wb_v[pl.ds(0, 16)]
    w1 = wb_v[pl.ds(16, 16)]
    bias_vec = wb_v[pl.ds(FACTOR, 16)]
    lanes = lax.iota(jnp.int32, 16)

    for j in range(NCHUNK):
        nxt = fire(j + 1) if j + 1 < NCHUNK else None
        pending[0].wait()
        pending[1].wait()
        pending = nxt
        buf = j % NBUF
        out_base = j * CHUNK

        @plsc.parallel_loop(0, CHUNK, step=16, unroll=2)
        def _block(i0):
            rv = i0 + lanes
            qa = (lax.shift_right_logical(
                idx_a[pl.ds(out_base + i0, 16)], 9) & 3) * FACTOR
            qb = (lax.shift_right_logical(
                idx_b[pl.ds(out_base + i0, 16)], 9) & 3) * FACTOR
            acc0 = bias_vec
            acc1 = jnp.zeros((16,), jnp.float32)
            for f in range(FACTOR):
                ga = plsc.load_gather(rows_a.at[buf], [rv, qa + f])
                gb = plsc.load_gather(rows_b.at[buf], [rv, qb + f])
                wf = w0[f] if f < 16 else w1[f - 16]
                prod = ga * gb * wf
                if f % 2 == 0:
                    acc0 = acc0 + prod
                else:
                    acc1 = acc1 + prod
            out_v[pl.ds(out_base + i0, 16)] = acc0 + acc1

    pltpu.sync_copy(out_v, out_hbm.at[pl.ds(base, B_PER_W)])


_sc_call = pl.kernel(
    _body,
    out_type=jax.ShapeDtypeStruct((BATCH,), jnp.float32),
    mesh=plsc.VectorSubcoreMesh(
        core_axis_name="c", subcore_axis_name="s",
        num_cores=NC, num_subcores=NS),
    scratch_types=[
        pltpu.VMEM((B_PER_W,), jnp.int32),
        pltpu.VMEM((B_PER_W,), jnp.int32),
        pltpu.VMEM((B_PER_W,), jnp.int32),
        pltpu.VMEM((B_PER_W,), jnp.int32),
        pltpu.VMEM((NBUF, CHUNK, LINE), jnp.float32),
        pltpu.VMEM((NBUF, CHUNK, LINE), jnp.float32),
        pltpu.VMEM((48,), jnp.float32),
        pltpu.VMEM((B_PER_W,), jnp.float32),
        pltpu.SemaphoreType.DMA,
        pltpu.SemaphoreType.DMA,
        pltpu.SemaphoreType.DMA,
    ],
    compiler_params=pltpu.CompilerParams(needs_layout_passes=False),
)


@jax.jit
def _run(table, fromk, tok, wb):
    lines = _tc_lines(table.T, jnp.eye(FACTOR, dtype=jnp.float32))
    return _sc_call(lines, fromk, tok, wb)


def kernel(fromk, tok, embed_k_GMF, predict_W, predict_b):
    wb = jnp.concatenate([
        predict_W.reshape(-1).astype(jnp.float32),
        jnp.broadcast_to(predict_b.astype(jnp.float32).reshape(-1)[:1], (16,)),
    ])
    return _run(embed_k_GMF, fromk.astype(jnp.int32), tok.astype(jnp.int32),
                wb)


# speedup vs baseline: 1.5786x; 1.5786x over previous
import jax
import jax.numpy as jnp
from jax import lax
from jax.experimental import pallas as pl
from jax.experimental.pallas import tpu as pltpu
from jax.experimental.pallas import tpu_sc as plsc

BATCH = 16384
FACTOR = 32
KROWS = 1000000
ROWS_PER_LINE = 4
NLINES = KROWS // ROWS_PER_LINE
LINE = ROWS_PER_LINE * FACTOR
NC = 2
NS = 16
NW = NC * NS
B_PER_W = BATCH // NW
CHUNK = 128
NCHUNK = B_PER_W // CHUNK
NBUF = 2

TCOLS = 16384
TLINES = TCOLS // ROWS_PER_LINE
TGRID = (KROWS + TCOLS - 1) // TCOLS
SB = TCOLS.bit_length() - 1
SQ = TLINES.bit_length() - 1


def _tbody(x_ref, eye_ref, o_ref):
    eye = eye_ref[...]
    parts = []
    for q in range(ROWS_PER_LINE):
        xq = x_ref[:, q * TLINES:(q + 1) * TLINES]
        parts.append(lax.dot_general(
            xq, eye, (((0,), (0,)), ((), ())),
            preferred_element_type=jnp.float32))
    o_ref[...] = jnp.concatenate(parts, axis=1)


_tc_lines = pl.pallas_call(
    _tbody,
    grid=(TGRID,),
    in_specs=[pl.BlockSpec((FACTOR, TCOLS), lambda i: (0, i)),
              pl.BlockSpec((FACTOR, FACTOR), lambda i: (0, 0))],
    out_specs=pl.BlockSpec((TLINES, LINE), lambda i: (i, 0)),
    out_shape=jax.ShapeDtypeStruct((TGRID * TLINES, LINE), jnp.float32),
)


def _body(table_hbm, fromk_hbm, tok_hbm, wb_hbm, out_hbm,
          idx_a, idx_b, line_a, line_b, rows_a, rows_b, wb_v, out_v,
          sem_i, sem_a, sem_b):
    wid = lax.axis_index("s") * NC + lax.axis_index("c")
    base = wid * B_PER_W

    cp_a = pltpu.async_copy(fromk_hbm.at[pl.ds(base, B_PER_W)], idx_a,
                            sem_i)
    cp_b = pltpu.async_copy(tok_hbm.at[pl.ds(base, B_PER_W)], idx_b,
                            sem_i)
    pltpu.sync_copy(wb_hbm, wb_v)
    cp_a.wait()
    cp_b.wait()

    @plsc.parallel_loop(0, B_PER_W, step=16, unroll=4)
    def _shift(k):
        iva = idx_a[pl.ds(k, 16)]
        ivb = idx_b[pl.ds(k, 16)]
        line_a[pl.ds(k, 16)] = (
            lax.shift_left(lax.shift_right_logical(iva, SB), SQ)
            + (iva & (TLINES - 1)))
        line_b[pl.ds(k, 16)] = (
            lax.shift_left(lax.shift_right_logical(ivb, SB), SQ)
            + (ivb & (TLINES - 1)))

    def fire(j):
        buf = j % NBUF
        ids = pl.ds(j * CHUNK, CHUNK)
        return (
            pltpu.async_copy(table_hbm.at[line_a.at[ids]], rows_a.at[buf],
                             sem_a),
            pltpu.async_copy(table_hbm.at[line_b.at[ids]], rows_b.at[buf],
                             sem_b),
        )

    pending = fire(0)

    w0 = wb_v[pl.ds(0, 16)]
    w1 = wb_v[pl.ds(16, 16)]
    bias_vec = wb_v[pl.ds(FACTOR, 16)]
    lanes = lax.iota(jnp.int32, 16)

    for j in range(NCHUNK):
        nxt = fire(j + 1) if j + 1 < NCHUNK else None
        pending[0].wait()
        pending[1].wait()
        pending = nxt
        buf = j % NBUF
        out_base = j * CHUNK

        @plsc.parallel_loop(0, CHUNK, step=16, unroll=2)
        def _block(i0):
            rv = i0 + lanes
            qa = (lax.shift_right_logical(
                idx_a[pl.ds(out_base + i0, 16)], SQ) & 3) * FACTOR
            qb = (lax.shift_right_logical(
                idx_b[pl.ds(out_base + i0, 16)], SQ) & 3) * FACTOR
            acc0 = bias_vec
            acc1 = jnp.zeros((16,), jnp.float32)
            for f in range(FACTOR):
                ga = plsc.load_gather(rows_a.at[buf], [rv, qa + f])
                gb = plsc.load_gather(rows_b.at[buf], [rv, qb + f])
                wf = w0[f] if f < 16 else w1[f - 16]
                prod = ga * gb * wf
                if f % 2 == 0:
                    acc0 = acc0 + prod
                else:
                    acc1 = acc1 + prod
            out_v[pl.ds(out_base + i0, 16)] = acc0 + acc1

    pltpu.sync_copy(out_v, out_hbm.at[pl.ds(base, B_PER_W)])


_sc_call = pl.kernel(
    _body,
    out_type=jax.ShapeDtypeStruct((BATCH,), jnp.float32),
    mesh=plsc.VectorSubcoreMesh(
        core_axis_name="c", subcore_axis_name="s",
        num_cores=NC, num_subcores=NS),
    scratch_types=[
        pltpu.VMEM((B_PER_W,), jnp.int32),
        pltpu.VMEM((B_PER_W,), jnp.int32),
        pltpu.VMEM((B_PER_W,), jnp.int32),
        pltpu.VMEM((B_PER_W,), jnp.int32),
        pltpu.VMEM((NBUF, CHUNK, LINE), jnp.float32),
        pltpu.VMEM((NBUF, CHUNK, LINE), jnp.float32),
        pltpu.VMEM((48,), jnp.float32),
        pltpu.VMEM((B_PER_W,), jnp.float32),
        pltpu.SemaphoreType.DMA,
        pltpu.SemaphoreType.DMA,
        pltpu.SemaphoreType.DMA,
    ],
    compiler_params=pltpu.CompilerParams(needs_layout_passes=False),
)


@jax.jit
def _run(table, fromk, tok, wb):
    lines = _tc_lines(table.T, jnp.eye(FACTOR, dtype=jnp.float32))
    return _sc_call(lines, fromk, tok, wb)


def kernel(fromk, tok, embed_k_GMF, predict_W, predict_b):
    wb = jnp.concatenate([
        predict_W.reshape(-1).astype(jnp.float32),
        jnp.broadcast_to(predict_b.astype(jnp.float32).reshape(-1)[:1], (16,)),
    ])
    return _run(embed_k_GMF, fromk.astype(jnp.int32), tok.astype(jnp.int32),
                wb)


# speedup vs baseline: 1.5874x; 1.0056x over previous
import jax
import jax.numpy as jnp
from jax import lax
from jax.experimental import pallas as pl
from jax.experimental.pallas import tpu as pltpu
from jax.experimental.pallas import tpu_sc as plsc

BATCH = 16384
FACTOR = 32
KROWS = 1000000
ROWS_PER_LINE = 4
NLINES = KROWS // ROWS_PER_LINE
LINE = ROWS_PER_LINE * FACTOR
NC = 2
NS = 16
NW = NC * NS
B_PER_W = BATCH // NW
CHUNK = 128
NCHUNK = B_PER_W // CHUNK
NBUF = 2

TCOLS = 32768
TLINES = TCOLS // ROWS_PER_LINE
TGRID = (KROWS + TCOLS - 1) // TCOLS
SB = TCOLS.bit_length() - 1
SQ = TLINES.bit_length() - 1


def _tbody(x_ref, eye_ref, o_ref):
    eye = eye_ref[...]
    parts = []
    for q in range(ROWS_PER_LINE):
        xq = x_ref[:, q * TLINES:(q + 1) * TLINES]
        parts.append(lax.dot_general(
            xq, eye, (((0,), (0,)), ((), ())),
            preferred_element_type=jnp.float32))
    o_ref[...] = jnp.concatenate(parts, axis=1)


_tc_lines = pl.pallas_call(
    _tbody,
    grid=(TGRID,),
    in_specs=[pl.BlockSpec((FACTOR, TCOLS), lambda i: (0, i)),
              pl.BlockSpec((FACTOR, FACTOR), lambda i: (0, 0))],
    out_specs=pl.BlockSpec((TLINES, LINE), lambda i: (i, 0)),
    out_shape=jax.ShapeDtypeStruct((TGRID * TLINES, LINE), jnp.float32),
)


def _body(table_hbm, fromk_hbm, tok_hbm, wb_hbm, out_hbm,
          idx_a, idx_b, line_a, line_b, rows_a, rows_b, wb_v, out_v,
          sem_i, sem_a, sem_b):
    wid = lax.axis_index("s") * NC + lax.axis_index("c")
    base = wid * B_PER_W

    cp_a = pltpu.async_copy(fromk_hbm.at[pl.ds(base, B_PER_W)], idx_a,
                            sem_i)
    cp_b = pltpu.async_copy(tok_hbm.at[pl.ds(base, B_PER_W)], idx_b,
                            sem_i)
    pltpu.sync_copy(wb_hbm, wb_v)
    cp_a.wait()
    cp_b.wait()

    @plsc.parallel_loop(0, B_PER_W, step=16, unroll=4)
    def _shift(k):
        iva = idx_a[pl.ds(k, 16)]
        ivb = idx_b[pl.ds(k, 16)]
        line_a[pl.ds(k, 16)] = (
            lax.shift_left(lax.shift_right_logical(iva, SB), SQ)
            + (iva & (TLINES - 1)))
        line_b[pl.ds(k, 16)] = (
            lax.shift_left(lax.shift_right_logical(ivb, SB), SQ)
            + (ivb & (TLINES - 1)))

    def fire(j):
        buf = j % NBUF
        ids = pl.ds(j * CHUNK, CHUNK)
        return (
            pltpu.async_copy(table_hbm.at[line_a.at[ids]], rows_a.at[buf],
                             sem_a),
            pltpu.async_copy(table_hbm.at[line_b.at[ids]], rows_b.at[buf],
                             sem_b),
        )

    pending = fire(0)

    w0 = wb_v[pl.ds(0, 16)]
    w1 = wb_v[pl.ds(16, 16)]
    bias_vec = wb_v[pl.ds(FACTOR, 16)]
    lanes = lax.iota(jnp.int32, 16)

    for j in range(NCHUNK):
        nxt = fire(j + 1) if j + 1 < NCHUNK else None
        pending[0].wait()
        pending[1].wait()
        pending = nxt
        buf = j % NBUF
        out_base = j * CHUNK

        @plsc.parallel_loop(0, CHUNK, step=16, unroll=2)
        def _block(i0):
            rv = i0 + lanes
            qa = (lax.shift_right_logical(
                idx_a[pl.ds(out_base + i0, 16)], SQ) & 3) * FACTOR
            qb = (lax.shift_right_logical(
                idx_b[pl.ds(out_base + i0, 16)], SQ) & 3) * FACTOR
            acc0 = bias_vec
            acc1 = jnp.zeros((16,), jnp.float32)
            for f in range(FACTOR):
                ga = plsc.load_gather(rows_a.at[buf], [rv, qa + f])
                gb = plsc.load_gather(rows_b.at[buf], [rv, qb + f])
                wf = w0[f] if f < 16 else w1[f - 16]
                prod = ga * gb * wf
                if f % 2 == 0:
                    acc0 = acc0 + prod
                else:
                    acc1 = acc1 + prod
            out_v[pl.ds(out_base + i0, 16)] = acc0 + acc1

    pltpu.sync_copy(out_v, out_hbm.at[pl.ds(base, B_PER_W)])


_sc_call = pl.kernel(
    _body,
    out_type=jax.ShapeDtypeStruct((BATCH,), jnp.float32),
    mesh=plsc.VectorSubcoreMesh(
        core_axis_name="c", subcore_axis_name="s",
        num_cores=NC, num_subcores=NS),
    scratch_types=[
        pltpu.VMEM((B_PER_W,), jnp.int32),
        pltpu.VMEM((B_PER_W,), jnp.int32),
        pltpu.VMEM((B_PER_W,), jnp.int32),
        pltpu.VMEM((B_PER_W,), jnp.int32),
        pltpu.VMEM((NBUF, CHUNK, LINE), jnp.float32),
        pltpu.VMEM((NBUF, CHUNK, LINE), jnp.float32),
        pltpu.VMEM((48,), jnp.float32),
        pltpu.VMEM((B_PER_W,), jnp.float32),
        pltpu.SemaphoreType.DMA,
        pltpu.SemaphoreType.DMA,
        pltpu.SemaphoreType.DMA,
    ],
    compiler_params=pltpu.CompilerParams(needs_layout_passes=False),
)


@jax.jit
def _run(table, fromk, tok, wb):
    lines = _tc_lines(table.T, jnp.eye(FACTOR, dtype=jnp.float32))
    return _sc_call(lines, fromk, tok, wb)


def kernel(fromk, tok, embed_k_GMF, predict_W, predict_b):
    wb = jnp.concatenate([
        predict_W.reshape(-1).astype(jnp.float32),
        jnp.broadcast_to(predict_b.astype(jnp.float32).reshape(-1)[:1], (16,)),
    ])
    return _run(embed_k_GMF, fromk.astype(jnp.int32), tok.astype(jnp.int32),
                wb)


# speedup vs baseline: 2.4036x; 1.5142x over previous
import jax
import jax.numpy as jnp
from jax import lax
from jax.experimental import pallas as pl
from jax.experimental.pallas import tpu as pltpu
from jax.experimental.pallas import tpu_sc as plsc

BATCH = 16384
FACTOR = 32
KROWS = 1000000
ROWS_PER_LINE = 4
NLINES = KROWS // ROWS_PER_LINE
LINE = ROWS_PER_LINE * FACTOR
NC = 2
NS = 16
NW = NC * NS
B_PER_W = BATCH // NW
CHUNK = 128
NCHUNK = B_PER_W // CHUNK
NBUF = 2

TCOLS = 32768
TLINES = TCOLS // ROWS_PER_LINE
TGRID = (KROWS + TCOLS - 1) // TCOLS
SB = TCOLS.bit_length() - 1
SQ = TLINES.bit_length() - 1


def _tbody(x_ref, eye_ref, o_ref):
    acc = None
    for q in range(ROWS_PER_LINE):
        xq = x_ref[:, q * TLINES:(q + 1) * TLINES]
        eq = eye_ref[q * FACTOR:(q + 1) * FACTOR, :]
        d = lax.dot_general(xq, eq, (((0,), (0,)), ((), ())),
                            preferred_element_type=jnp.float32)
        acc = d if acc is None else acc + d
    o_ref[...] = acc


_tc_lines = pl.pallas_call(
    _tbody,
    grid=(TGRID,),
    in_specs=[pl.BlockSpec((FACTOR, TCOLS), lambda i: (0, i)),
              pl.BlockSpec((LINE, LINE), lambda i: (0, 0))],
    out_specs=pl.BlockSpec((TLINES, LINE), lambda i: (i, 0)),
    out_shape=jax.ShapeDtypeStruct((TGRID * TLINES, LINE), jnp.float32),
)


def _body(table_hbm, fromk_hbm, tok_hbm, wb_hbm, out_hbm,
          idx_a, idx_b, line_a, line_b, rows_a, rows_b, wb_v, out_v,
          sem_i, sem_a, sem_b):
    wid = lax.axis_index("s") * NC + lax.axis_index("c")
    base = wid * B_PER_W

    cp_a = pltpu.async_copy(fromk_hbm.at[pl.ds(base, B_PER_W)], idx_a,
                            sem_i)
    cp_b = pltpu.async_copy(tok_hbm.at[pl.ds(base, B_PER_W)], idx_b,
                            sem_i)
    pltpu.sync_copy(wb_hbm, wb_v)
    cp_a.wait()
    cp_b.wait()

    @plsc.parallel_loop(0, B_PER_W, step=16, unroll=4)
    def _shift(k):
        iva = idx_a[pl.ds(k, 16)]
        ivb = idx_b[pl.ds(k, 16)]
        line_a[pl.ds(k, 16)] = (
            lax.shift_left(lax.shift_right_logical(iva, SB), SQ)
            + (iva & (TLINES - 1)))
        line_b[pl.ds(k, 16)] = (
            lax.shift_left(lax.shift_right_logical(ivb, SB), SQ)
            + (ivb & (TLINES - 1)))

    def fire(j):
        buf = j % NBUF
        ids = pl.ds(j * CHUNK, CHUNK)
        return (
            pltpu.async_copy(table_hbm.at[line_a.at[ids]], rows_a.at[buf],
                             sem_a),
            pltpu.async_copy(table_hbm.at[line_b.at[ids]], rows_b.at[buf],
                             sem_b),
        )

    pending = fire(0)

    w0 = wb_v[pl.ds(0, 16)]
    w1 = wb_v[pl.ds(16, 16)]
    bias_vec = wb_v[pl.ds(FACTOR, 16)]
    lanes = lax.iota(jnp.int32, 16)

    for j in range(NCHUNK):
        nxt = fire(j + 1) if j + 1 < NCHUNK else None
        pending[0].wait()
        pending[1].wait()
        pending = nxt
        buf = j % NBUF
        out_base = j * CHUNK

        @plsc.parallel_loop(0, CHUNK, step=16, unroll=2)
        def _block(i0):
            rv = i0 + lanes
            qa = (lax.shift_right_logical(
                idx_a[pl.ds(out_base + i0, 16)], SQ) & 3) * FACTOR
            qb = (lax.shift_right_logical(
                idx_b[pl.ds(out_base + i0, 16)], SQ) & 3) * FACTOR
            acc0 = bias_vec
            acc1 = jnp.zeros((16,), jnp.float32)
            for f in range(FACTOR):
                ga = plsc.load_gather(rows_a.at[buf], [rv, qa + f])
                gb = plsc.load_gather(rows_b.at[buf], [rv, qb + f])
                wf = w0[f] if f < 16 else w1[f - 16]
                prod = ga * gb * wf
                if f % 2 == 0:
                    acc0 = acc0 + prod
                else:
                    acc1 = acc1 + prod
            out_v[pl.ds(out_base + i0, 16)] = acc0 + acc1

    pltpu.sync_copy(out_v, out_hbm.at[pl.ds(base, B_PER_W)])


_sc_call = pl.kernel(
    _body,
    out_type=jax.ShapeDtypeStruct((BATCH,), jnp.float32),
    mesh=plsc.VectorSubcoreMesh(
        core_axis_name="c", subcore_axis_name="s",
        num_cores=NC, num_subcores=NS),
    scratch_types=[
        pltpu.VMEM((B_PER_W,), jnp.int32),
        pltpu.VMEM((B_PER_W,), jnp.int32),
        pltpu.VMEM((B_PER_W,), jnp.int32),
        pltpu.VMEM((B_PER_W,), jnp.int32),
        pltpu.VMEM((NBUF, CHUNK, LINE), jnp.float32),
        pltpu.VMEM((NBUF, CHUNK, LINE), jnp.float32),
        pltpu.VMEM((48,), jnp.float32),
        pltpu.VMEM((B_PER_W,), jnp.float32),
        pltpu.SemaphoreType.DMA,
        pltpu.SemaphoreType.DMA,
        pltpu.SemaphoreType.DMA,
    ],
    compiler_params=pltpu.CompilerParams(needs_layout_passes=False),
)


@jax.jit
def _run(table, fromk, tok, wb):
    lines = _tc_lines(table.T, jnp.eye(LINE, dtype=jnp.float32))
    return _sc_call(lines, fromk, tok, wb)


def kernel(fromk, tok, embed_k_GMF, predict_W, predict_b):
    wb = jnp.concatenate([
        predict_W.reshape(-1).astype(jnp.float32),
        jnp.broadcast_to(predict_b.astype(jnp.float32).reshape(-1)[:1], (16,)),
    ])
    return _run(embed_k_GMF, fromk.astype(jnp.int32), tok.astype(jnp.int32),
                wb)
